# baseline (speedup 1.0000x reference)
import functools

import jax
import jax.numpy as jnp
from jax import lax
from jax.experimental import pallas as pl
from jax.experimental.pallas import tpu as pltpu
from jax.experimental.pallas import tpu_sc as plsc

CROP_H = 7
CROP_W = 7
L = 16


def _sc_crop_and_resize(table, boxes_pad, B, C, H, W, npb, nw):
    npad = npb * nw
    mesh = plsc.VectorSubcoreMesh(core_axis_name="c", subcore_axis_name="s")
    nc = mesh.num_cores

    @functools.partial(
        pl.kernel,
        out_type=jax.ShapeDtypeStruct((npad, C, 4 * L), jnp.float32),
        mesh=mesh,
        scratch_types=[
            pltpu.VMEM((npb, L), jnp.float32),
            pltpu.VMEM((4 * L,), jnp.int32),
            pltpu.VMEM((4 * L, C), jnp.float32),
            pltpu.VMEM((C, 4 * L), jnp.float32),
            pltpu.SemaphoreType.DMA,
        ],
        compiler_params=pltpu.CompilerParams(needs_layout_passes=False),
    )
    def k(table_hbm, boxes_hbm, out_hbm, boxes_v, idx_v, rows_v, outT_v, sem):
        wid = lax.axis_index("s") * nc + lax.axis_index("c")
        pltpu.sync_copy(boxes_hbm.at[pl.ds(wid * npb, npb)], boxes_v)
        lane = lax.iota(jnp.int32, L)
        fH = jnp.float32(H - 1)
        fW = jnp.float32(W - 1)

        def box_body(b, carry):
            bsplat = jnp.broadcast_to(b, (L,)).astype(jnp.int32)
            zero = jnp.zeros((L,), jnp.int32)
            y1 = plsc.load_gather(boxes_v, [bsplat, zero])
            x1 = plsc.load_gather(boxes_v, [bsplat, zero + 1])
            y2 = plsc.load_gather(boxes_v, [bsplat, zero + 2])
            x2 = plsc.load_gather(boxes_v, [bsplat, zero + 3])
            bif = plsc.load_gather(boxes_v, [bsplat, zero + 4])
            bi = jnp.clip(bif.astype(jnp.int32), 0, B - 1)

            for g in range(4):
                cell = g * L + lane
                iy = (cell * 147) >> 10
                ix = cell - iy * CROP_W
                ys = iy.astype(jnp.float32) / jnp.float32(CROP_H - 1)
                xs = ix.astype(jnp.float32) / jnp.float32(CROP_W - 1)
                in_y = (y1 + ys * (y2 - y1)) * fH
                in_x = (x1 + xs * (x2 - x1)) * fW
                valid = ((in_y >= 0.0) & (in_y <= fH)
                         & (in_x >= 0.0) & (in_x <= fW))

                ty0 = in_y.astype(jnp.int32)
                tf0 = ty0.astype(jnp.float32)
                neg = in_y < tf0
                fy_i = jnp.where(neg, ty0 - 1, ty0)
                fy_f = jnp.where(neg, tf0 - 1.0, tf0)
                ly = in_y - fy_f
                ti = jnp.clip(fy_i, 0, H - 1)
                bo = jnp.clip(fy_i + 1, 0, H - 1)

                tx0 = in_x.astype(jnp.int32)
                xf0 = tx0.astype(jnp.float32)
                negx = in_x < xf0
                fx_i = jnp.where(negx, tx0 - 1, tx0)
                fx_f = jnp.where(negx, xf0 - 1.0, xf0)
                lx = in_x - fx_f
                li = jnp.clip(fx_i, 0, W - 1)
                ri = jnp.clip(fx_i + 1, 0, W - 1)

                vf = jnp.where(valid, 1.0, 0.0).astype(jnp.float32)
                omy = 1.0 - ly
                omx = 1.0 - lx
                w_tl = omy * omx * vf
                w_tr = omy * lx * vf
                w_bl = ly * omx * vf
                w_br = ly * lx * vf

                trow = bi * (H * W) + ti * W
                brow = bi * (H * W) + bo * W
                idx_v[pl.ds(0, L)] = trow + li
                idx_v[pl.ds(L, L)] = trow + ri
                idx_v[pl.ds(2 * L, L)] = brow + li
                idx_v[pl.ds(3 * L, L)] = brow + ri

                pltpu.async_copy(table_hbm.at[idx_v], rows_v, sem).wait()

                def dbody(d, c, w_tl=w_tl, w_tr=w_tr, w_bl=w_bl, w_br=w_br,
                          g=g):
                    dd = jnp.broadcast_to(d, (L,)).astype(jnp.int32)
                    tl = plsc.load_gather(rows_v, [lane, dd])
                    tr = plsc.load_gather(rows_v, [lane + L, dd])
                    bl = plsc.load_gather(rows_v, [lane + 2 * L, dd])
                    br = plsc.load_gather(rows_v, [lane + 3 * L, dd])
                    val = w_tl * tl + w_tr * tr + w_bl * bl + w_br * br
                    outT_v[d, pl.ds(g * L, L)] = val
                    return c

                lax.fori_loop(0, C, dbody, 0)

            n = wid * npb + b
            pltpu.sync_copy(outT_v, out_hbm.at[n])
            return carry

        lax.fori_loop(0, npb, box_body, 0)

    return k(table, boxes_pad)


def kernel(image, boxes, box_ind):
    B, C, H, W = image.shape
    N = boxes.shape[0]
    nw = 32
    npb = -(-N // (nw * 8)) * 8
    npad = npb * nw

    table = jnp.transpose(image, (0, 2, 3, 1)).reshape(B * H * W, C)
    boxes5 = jnp.concatenate(
        [boxes, box_ind[:, None].astype(jnp.float32)], axis=1)
    boxes_pad = jnp.zeros((npad, L), jnp.float32).at[:N, :5].set(boxes5)

    out = _sc_crop_and_resize(table, boxes_pad, B, C, H, W, npb, nw)
    return out[:N, :, : CROP_H * CROP_W].reshape(N, C, CROP_H, CROP_W)


# speedup vs baseline: 3.1287x; 3.1287x over previous
import functools

import jax
import jax.numpy as jnp
from jax import lax
from jax.experimental import pallas as pl
from jax.experimental.pallas import tpu as pltpu
from jax.experimental.pallas import tpu_sc as plsc

CROP_H = 7
CROP_W = 7
L = 16


def _sc_crop_and_resize(table, boxes_pad, B, C, H, W, npb, nw):
    npad = npb * nw
    ng = npb * 4
    nvr = C // L
    mesh = plsc.VectorSubcoreMesh(core_axis_name="c", subcore_axis_name="s")
    nc = mesh.num_cores

    @functools.partial(
        pl.kernel,
        out_type=jax.ShapeDtypeStruct((npad * 4, L, C), jnp.float32),
        mesh=mesh,
        scratch_types=[
            pltpu.VMEM((npb, L), jnp.float32),
            pltpu.VMEM((4 * L,), jnp.int32),
            pltpu.VMEM((4 * L,), jnp.int32),
            pltpu.VMEM((4 * L, C), jnp.float32),
            pltpu.VMEM((4 * L, C), jnp.float32),
            pltpu.VMEM((4, L), jnp.float32),
            pltpu.VMEM((4, L), jnp.float32),
            pltpu.VMEM((L, C), jnp.float32),
            pltpu.SemaphoreType.DMA((2,)),
        ],
        compiler_params=pltpu.CompilerParams(needs_layout_passes=False),
    )
    def k(table_hbm, boxes_hbm, out_hbm, boxes_v,
          idx0, idx1, rows0, rows1, wb0, wb1, out_v, sems):
        wid = lax.axis_index("s") * nc + lax.axis_index("c")
        pltpu.sync_copy(boxes_hbm.at[pl.ds(wid * npb, npb)], boxes_v)
        lane = lax.iota(jnp.int32, L)
        zero = jnp.zeros((L,), jnp.int32)
        fH = jnp.float32(H - 1)
        fW = jnp.float32(W - 1)
        bufs = ((idx0, rows0, wb0, sems.at[0]), (idx1, rows1, wb1, sems.at[1]))

        def stage(gi, buf):
            idxr, rowsr, wbr, sem = buf
            b = gi >> 2
            g = gi & 3
            bs = jnp.broadcast_to(b, (L,))
            y1 = plsc.load_gather(boxes_v, [bs, zero])
            x1 = plsc.load_gather(boxes_v, [bs, zero + 1])
            y2 = plsc.load_gather(boxes_v, [bs, zero + 2])
            x2 = plsc.load_gather(boxes_v, [bs, zero + 3])
            bif = plsc.load_gather(boxes_v, [bs, zero + 4])
            bi = jnp.clip(bif.astype(jnp.int32), 0, B - 1)

            cell = jnp.broadcast_to(g * L, (L,)) + lane
            iy = (cell * 147) >> 10
            ix = cell - iy * CROP_W
            ys = iy.astype(jnp.float32) / jnp.float32(CROP_H - 1)
            xs = ix.astype(jnp.float32) / jnp.float32(CROP_W - 1)
            in_y = (y1 + ys * (y2 - y1)) * fH
            in_x = (x1 + xs * (x2 - x1)) * fW
            valid = ((in_y >= 0.0) & (in_y <= fH)
                     & (in_x >= 0.0) & (in_x <= fW))

            ty0 = in_y.astype(jnp.int32)
            tf0 = ty0.astype(jnp.float32)
            neg = in_y < tf0
            fy_i = jnp.where(neg, ty0 - 1, ty0)
            fy_f = jnp.where(neg, tf0 - 1.0, tf0)
            ly = in_y - fy_f
            ti = jnp.clip(fy_i, 0, H - 1)
            bo = jnp.clip(fy_i + 1, 0, H - 1)

            tx0 = in_x.astype(jnp.int32)
            xf0 = tx0.astype(jnp.float32)
            negx = in_x < xf0
            fx_i = jnp.where(negx, tx0 - 1, tx0)
            fx_f = jnp.where(negx, xf0 - 1.0, xf0)
            lx = in_x - fx_f
            li = jnp.clip(fx_i, 0, W - 1)
            ri = jnp.clip(fx_i + 1, 0, W - 1)

            vf = jnp.where(valid, 1.0, 0.0).astype(jnp.float32)
            omy = 1.0 - ly
            omx = 1.0 - lx
            wbr[0, :] = omy * omx * vf
            wbr[1, :] = omy * lx * vf
            wbr[2, :] = ly * omx * vf
            wbr[3, :] = ly * lx * vf

            trow = bi * (H * W) + ti * W
            brow = bi * (H * W) + bo * W
            idxr[pl.ds(0, L)] = trow + li
            idxr[pl.ds(L, L)] = trow + ri
            idxr[pl.ds(2 * L, L)] = brow + li
            idxr[pl.ds(3 * L, L)] = brow + ri

            pltpu.async_copy(table_hbm.at[idxr], rowsr, sem)

        def consume(gi, buf):
            idxr, rowsr, wbr, sem = buf
            pltpu.make_async_copy(table_hbm.at[idxr], rowsr, sem).wait()

            def cell_body(j, c):
                js = jnp.broadcast_to(j, (L,))
                wtl = plsc.load_gather(wbr, [zero, js])
                wtr = plsc.load_gather(wbr, [zero + 1, js])
                wbl = plsc.load_gather(wbr, [zero + 2, js])
                wbr_ = plsc.load_gather(wbr, [zero + 3, js])
                for kk in range(nvr):
                    sl = pl.ds(kk * L, L)
                    val = (wtl * rowsr[j, sl] + wtr * rowsr[j + L, sl]
                           + wbl * rowsr[j + 2 * L, sl]
                           + wbr_ * rowsr[j + 3 * L, sl])
                    out_v[j, sl] = val
                return c

            lax.fori_loop(0, L, cell_body, 0)
            pltpu.sync_copy(out_v, out_hbm.at[wid * ng + gi])

        def outer(i, c):
            stage(i, bufs[0])
            consume(i, bufs[0])
            return c

        lax.fori_loop(0, ng, outer, 0)

    return k(table, boxes_pad)


def kernel(image, boxes, box_ind):
    B, C, H, W = image.shape
    N = boxes.shape[0]
    nw = 32
    npb = -(-N // (nw * 8)) * 8
    npad = npb * nw

    table = jnp.transpose(image, (0, 2, 3, 1)).reshape(B * H * W, C)
    boxes5 = jnp.concatenate(
        [boxes, box_ind[:, None].astype(jnp.float32)], axis=1)
    boxes_pad = jnp.zeros((npad, L), jnp.float32).at[:N, :5].set(boxes5)

    out = _sc_crop_and_resize(table, boxes_pad, B, C, H, W, npb, nw)
    out = out.reshape(npad, 4 * L, C)[:N, : CROP_H * CROP_W, :]
    return jnp.transpose(out, (0, 2, 1)).reshape(N, C, CROP_H, CROP_W)
